# baseline (speedup 1.0000x reference)
import jax
import jax.numpy as jnp
from jax import lax
from jax.experimental import pallas as pl
from jax.experimental.pallas import tpu as pltpu
from jax.experimental.pallas import tpu_sc as plsc

_N = 10000
_E = 320000
_D = 128
_G = 64
_DO = 16

_NC = 2
_NS = 16
_NW = _NC * _NS
_K = 80
_CH = 125
_EPW = _CH * _K
_NP = 10240
_RPT = _NP // _NS


def _segsum_body(t_hbm, s_hbm, d_hbm, out_hbm, didx, sidx, rows0, rows1,
                 acc, dsem, g0sem, g1sem, s0sem, s1sem):
    c = lax.axis_index("c")
    s = lax.axis_index("s")
    wid = c * _NS + s

    cp_d = pltpu.async_copy(d_hbm.at[wid], didx, dsem)
    cp_s = pltpu.async_copy(s_hbm.at[wid], sidx, dsem)

    def zrow(r, carry):
        def zcol(j, carry2):
            rows0[r, pl.ds(j * 16, 16)] = jnp.zeros((16,), jnp.float32)
            return carry2
        return lax.fori_loop(0, _D // 16, zcol, carry)
    lax.fori_loop(0, _K, zrow, 0)
    for j in range(_RPT // _K):
        pltpu.sync_copy(rows0, acc.at[pl.ds(s * _RPT + j * _K, _K)])
    cp_d.wait()
    cp_s.wait()
    plsc.subcore_barrier()

    pltpu.async_copy(t_hbm.at[sidx.at[pl.ds(0, _K)]], rows0, g0sem)

    def body(j, carry):
        i0 = 2 * j
        i1 = 2 * j + 1

        pltpu.async_copy(t_hbm.at[sidx.at[pl.ds(i1 * _K, _K)]], rows1, g1sem)
        pltpu.make_async_copy(t_hbm.at[sidx.at[pl.ds(i0 * _K, _K)]], rows0,
                              g0sem).wait()
        pltpu.async_copy(rows0, acc.at[didx.at[i0]], s0sem, add=True)

        pltpu.make_async_copy(t_hbm.at[sidx.at[pl.ds(i1 * _K, _K)]], rows1,
                              g1sem).wait()
        pltpu.async_copy(rows1, acc.at[didx.at[i1]], s1sem, add=True)

        pltpu.make_async_copy(rows0, acc.at[didx.at[i0]], s0sem).wait()

        @pl.when(i0 + 2 < _CH)
        def _g0():
            pltpu.async_copy(t_hbm.at[sidx.at[pl.ds((i0 + 2) * _K, _K)]],
                             rows0, g0sem)

        pltpu.make_async_copy(rows1, acc.at[didx.at[i1]], s1sem).wait()

        return carry
    lax.fori_loop(0, _CH // 2, body, 0)

    last = _CH - 1
    pltpu.make_async_copy(t_hbm.at[sidx.at[pl.ds(last * _K, _K)]], rows0,
                          g0sem).wait()
    pltpu.sync_copy(rows0, acc.at[didx.at[last]], add=True)

    plsc.subcore_barrier()
    pltpu.sync_copy(acc.at[pl.ds(s * _RPT, _RPT)],
                    out_hbm.at[pl.ds(c * _NP + s * _RPT, _RPT)])


_segsum = pl.kernel(
    _segsum_body,
    out_type=jax.ShapeDtypeStruct((_NC * _NP, _D), jnp.float32),
    mesh=plsc.VectorSubcoreMesh(core_axis_name="c", subcore_axis_name="s"),
    scratch_types=[
        pltpu.VMEM((_CH, _K), jnp.int32),
        pltpu.VMEM((_EPW,), jnp.int32),
        pltpu.VMEM((_K, _D), jnp.float32),
        pltpu.VMEM((_K, _D), jnp.float32),
        pltpu.VMEM_SHARED((_NP, _D), jnp.float32),
        pltpu.SemaphoreType.DMA,
        pltpu.SemaphoreType.DMA,
        pltpu.SemaphoreType.DMA,
        pltpu.SemaphoreType.DMA,
        pltpu.SemaphoreType.DMA,
    ],
)

_BLK = 1000


def _mm_body(x_ref, w_ref, o_ref):
    o_ref[...] = jnp.dot(x_ref[...], w_ref[...],
                         preferred_element_type=jnp.float32)


def _mm(x, w):
    return pl.pallas_call(
        _mm_body,
        grid=(_N // _BLK,),
        in_specs=[pl.BlockSpec((_BLK, _D), lambda i: (i, 0)),
                  pl.BlockSpec((_D, _D), lambda i: (0, 0))],
        out_specs=pl.BlockSpec((_BLK, _D), lambda i: (i, 0)),
        out_shape=jax.ShapeDtypeStruct((_N, _D), jnp.float32),
    )(x, w)


def _fused_body(t_ref, p0_ref, p1_ref, b_ref, w_ref, o_ref):
    h = jnp.maximum(t_ref[...] + p0_ref[...] + p1_ref[...] + b_ref[...], 0.0)
    o_ref[...] = jnp.dot(h, w_ref[...], preferred_element_type=jnp.float32)


def _fused(t, p0, p1, b, w):
    return pl.pallas_call(
        _fused_body,
        grid=(_N // _BLK,),
        in_specs=[pl.BlockSpec((_BLK, _D), lambda i: (i, 0)),
                  pl.BlockSpec((_BLK, _D), lambda i: (i, 0)),
                  pl.BlockSpec((_BLK, _D), lambda i: (i, 0)),
                  pl.BlockSpec((1, _D), lambda i: (0, 0)),
                  pl.BlockSpec((_D, _D), lambda i: (0, 0))],
        out_specs=pl.BlockSpec((_BLK, _D), lambda i: (i, 0)),
        out_shape=jax.ShapeDtypeStruct((_N, _D), jnp.float32),
    )(t, p0, p1, b, w)


def _pool_body(t_ref, p0_ref, p1_ref, b_ref, batch_ref, wfc_ref, bfc_ref,
               o_ref, sums, cnt):
    i = pl.program_id(0)

    @pl.when(i == 0)
    def _init():
        sums[...] = jnp.zeros_like(sums)
        cnt[...] = jnp.zeros_like(cnt)

    h = jnp.maximum(t_ref[...] + p0_ref[...] + p1_ref[...] + b_ref[...], 0.0)
    gids = lax.broadcasted_iota(jnp.int32, (_BLK, _G), 1)
    oh = (batch_ref[...] == gids).astype(jnp.float32)
    sums[...] += lax.dot_general(oh, h, (((0,), (0,)), ((), ())),
                                 preferred_element_type=jnp.float32)
    cnt[...] += lax.dot_general(oh, jnp.ones((_BLK, _D), jnp.float32),
                                (((0,), (0,)), ((), ())),
                                preferred_element_type=jnp.float32)

    @pl.when(i == _N // _BLK - 1)
    def _fin():
        pooled = sums[...] / jnp.maximum(cnt[...], 1.0)
        o_ref[...] = jnp.dot(pooled, wfc_ref[...],
                             preferred_element_type=jnp.float32) + bfc_ref[...]


def _pool(t, p0, p1, b, batch2, wfc, bfc):
    return pl.pallas_call(
        _pool_body,
        grid=(_N // _BLK,),
        in_specs=[pl.BlockSpec((_BLK, _D), lambda i: (i, 0)),
                  pl.BlockSpec((_BLK, _D), lambda i: (i, 0)),
                  pl.BlockSpec((_BLK, _D), lambda i: (i, 0)),
                  pl.BlockSpec((1, _D), lambda i: (0, 0)),
                  pl.BlockSpec((_BLK, 1), lambda i: (i, 0)),
                  pl.BlockSpec((_D, _DO), lambda i: (0, 0)),
                  pl.BlockSpec((1, _DO), lambda i: (0, 0))],
        out_specs=pl.BlockSpec((_G, _DO), lambda i: (0, 0)),
        out_shape=jax.ShapeDtypeStruct((_G, _DO), jnp.float32),
        scratch_shapes=[pltpu.VMEM((_G, _D), jnp.float32),
                        pltpu.VMEM((_G, _D), jnp.float32)],
    )(t, p0, p1, b, batch2, wfc, bfc)


def kernel(x, edge_index, batch, W1, b1, W2, b2, W3, b3, Wfc, bfc):
    src2 = edge_index[0].reshape(_NW, _EPW)
    dst2 = edge_index[1].reshape(_NW, _CH, _K)
    batch2 = batch.reshape(_N, 1)
    b1r = b1.reshape(1, _D)
    b2r = b2.reshape(1, _D)
    b3r = b3.reshape(1, _D)
    bfcr = bfc.reshape(1, _DO)

    t1 = _mm(x, W1)
    s1 = _segsum(t1, src2, dst2)
    t2 = _fused(t1, s1[:_N], s1[_NP:_NP + _N], b1r, W2)
    s2 = _segsum(t2, src2, dst2)
    t3 = _fused(t2, s2[:_N], s2[_NP:_NP + _N], b2r, W3)
    s3 = _segsum(t3, src2, dst2)
    return _pool(t3, s3[:_N], s3[_NP:_NP + _N], b3r, batch2, Wfc, bfcr)


# speedup vs baseline: 1.2404x; 1.2404x over previous
import jax
import jax.numpy as jnp
from jax import lax
from jax.experimental import pallas as pl
from jax.experimental.pallas import tpu as pltpu
from jax.experimental.pallas import tpu_sc as plsc

_N = 10000
_E = 320000
_D = 128
_G = 64
_DO = 16

_NC = 2
_NS = 16
_NW = _NC * _NS
_K = 80
_CH = 125
_EPW = _CH * _K
_NP = 10240
_RPT = _NP // _NS


def _segsum_body(t_hbm, s_hbm, d_hbm, out_hbm, didx, sidx, rows0, rows1,
                 acc, dsem, g0sem, g1sem):
    c = lax.axis_index("c")
    s = lax.axis_index("s")
    wid = c * _NS + s

    cp_d = pltpu.async_copy(d_hbm.at[wid], didx, dsem)
    cp_s = pltpu.async_copy(s_hbm.at[wid], sidx, dsem)

    def zrow(r, carry):
        for j in range(_D // 16):
            rows0[r, pl.ds(j * 16, 16)] = jnp.zeros((16,), jnp.float32)
        return carry
    lax.fori_loop(0, _K, zrow, 0)
    for j in range(_RPT // _K):
        pltpu.sync_copy(rows0, acc.at[pl.ds(s * _RPT + j * _K, _K)])
    cp_d.wait()
    cp_s.wait()
    plsc.subcore_barrier()

    pltpu.async_copy(t_hbm.at[sidx.at[pl.ds(0, _K)]], rows0, g0sem)

    def body(j, carry):
        i0 = 2 * j
        i1 = 2 * j + 1

        pltpu.async_copy(t_hbm.at[sidx.at[pl.ds(i1 * _K, _K)]], rows1, g1sem)
        pltpu.make_async_copy(t_hbm.at[sidx.at[pl.ds(i0 * _K, _K)]], rows0,
                              g0sem).wait()
        pltpu.sync_copy(rows0, acc.at[didx.at[i0]], add=True)

        @pl.when(i0 + 2 < _CH)
        def _g0():
            pltpu.async_copy(t_hbm.at[sidx.at[pl.ds((i0 + 2) * _K, _K)]],
                             rows0, g0sem)

        pltpu.make_async_copy(t_hbm.at[sidx.at[pl.ds(i1 * _K, _K)]], rows1,
                              g1sem).wait()
        pltpu.sync_copy(rows1, acc.at[didx.at[i1]], add=True)

        return carry
    lax.fori_loop(0, _CH // 2, body, 0)

    last = _CH - 1
    pltpu.make_async_copy(t_hbm.at[sidx.at[pl.ds(last * _K, _K)]], rows0,
                          g0sem).wait()
    pltpu.sync_copy(rows0, acc.at[didx.at[last]], add=True)

    plsc.subcore_barrier()
    pltpu.sync_copy(acc.at[pl.ds(s * _RPT, _RPT)],
                    out_hbm.at[pl.ds(c * _NP + s * _RPT, _RPT)])


_segsum = pl.kernel(
    _segsum_body,
    out_type=jax.ShapeDtypeStruct((_NC * _NP, _D), jnp.float32),
    mesh=plsc.VectorSubcoreMesh(core_axis_name="c", subcore_axis_name="s"),
    scratch_types=[
        pltpu.VMEM((_CH, _K), jnp.int32),
        pltpu.VMEM((_EPW,), jnp.int32),
        pltpu.VMEM((_K, _D), jnp.float32),
        pltpu.VMEM((_K, _D), jnp.float32),
        pltpu.VMEM_SHARED((_NP, _D), jnp.float32),
        pltpu.SemaphoreType.DMA,
        pltpu.SemaphoreType.DMA,
        pltpu.SemaphoreType.DMA,
    ],
)

_BLK = 1000


def _mm_body(x_ref, w_ref, o_ref):
    o_ref[...] = jnp.dot(x_ref[...], w_ref[...],
                         preferred_element_type=jnp.float32)


def _mm(x, w):
    return pl.pallas_call(
        _mm_body,
        grid=(_N // _BLK,),
        in_specs=[pl.BlockSpec((_BLK, _D), lambda i: (i, 0)),
                  pl.BlockSpec((_D, _D), lambda i: (0, 0))],
        out_specs=pl.BlockSpec((_BLK, _D), lambda i: (i, 0)),
        out_shape=jax.ShapeDtypeStruct((_N, _D), jnp.float32),
    )(x, w)


def _fused_body(t_ref, p0_ref, p1_ref, b_ref, w_ref, o_ref):
    h = jnp.maximum(t_ref[...] + p0_ref[...] + p1_ref[...] + b_ref[...], 0.0)
    o_ref[...] = jnp.dot(h, w_ref[...], preferred_element_type=jnp.float32)


def _fused(t, p0, p1, b, w):
    return pl.pallas_call(
        _fused_body,
        grid=(_N // _BLK,),
        in_specs=[pl.BlockSpec((_BLK, _D), lambda i: (i, 0)),
                  pl.BlockSpec((_BLK, _D), lambda i: (i, 0)),
                  pl.BlockSpec((_BLK, _D), lambda i: (i, 0)),
                  pl.BlockSpec((1, _D), lambda i: (0, 0)),
                  pl.BlockSpec((_D, _D), lambda i: (0, 0))],
        out_specs=pl.BlockSpec((_BLK, _D), lambda i: (i, 0)),
        out_shape=jax.ShapeDtypeStruct((_N, _D), jnp.float32),
    )(t, p0, p1, b, w)


def _pool_body(t_ref, p0_ref, p1_ref, b_ref, batch_ref, wfc_ref, bfc_ref,
               o_ref, sums, cnt):
    i = pl.program_id(0)

    @pl.when(i == 0)
    def _init():
        sums[...] = jnp.zeros_like(sums)
        cnt[...] = jnp.zeros_like(cnt)

    h = jnp.maximum(t_ref[...] + p0_ref[...] + p1_ref[...] + b_ref[...], 0.0)
    gids = lax.broadcasted_iota(jnp.int32, (_BLK, _G), 1)
    oh = (batch_ref[...] == gids).astype(jnp.float32)
    sums[...] += lax.dot_general(oh, h, (((0,), (0,)), ((), ())),
                                 preferred_element_type=jnp.float32)
    cnt[...] += lax.dot_general(oh, jnp.ones((_BLK, _D), jnp.float32),
                                (((0,), (0,)), ((), ())),
                                preferred_element_type=jnp.float32)

    @pl.when(i == _N // _BLK - 1)
    def _fin():
        pooled = sums[...] / jnp.maximum(cnt[...], 1.0)
        o_ref[...] = jnp.dot(pooled, wfc_ref[...],
                             preferred_element_type=jnp.float32) + bfc_ref[...]


def _pool(t, p0, p1, b, batch2, wfc, bfc):
    return pl.pallas_call(
        _pool_body,
        grid=(_N // _BLK,),
        in_specs=[pl.BlockSpec((_BLK, _D), lambda i: (i, 0)),
                  pl.BlockSpec((_BLK, _D), lambda i: (i, 0)),
                  pl.BlockSpec((_BLK, _D), lambda i: (i, 0)),
                  pl.BlockSpec((1, _D), lambda i: (0, 0)),
                  pl.BlockSpec((_BLK, 1), lambda i: (i, 0)),
                  pl.BlockSpec((_D, _DO), lambda i: (0, 0)),
                  pl.BlockSpec((1, _DO), lambda i: (0, 0))],
        out_specs=pl.BlockSpec((_G, _DO), lambda i: (0, 0)),
        out_shape=jax.ShapeDtypeStruct((_G, _DO), jnp.float32),
        scratch_shapes=[pltpu.VMEM((_G, _D), jnp.float32),
                        pltpu.VMEM((_G, _D), jnp.float32)],
    )(t, p0, p1, b, batch2, wfc, bfc)


def kernel(x, edge_index, batch, W1, b1, W2, b2, W3, b3, Wfc, bfc):
    src2 = edge_index[0].reshape(_NW, _EPW)
    dst2 = edge_index[1].reshape(_NW, _CH, _K)
    batch2 = batch.reshape(_N, 1)
    b1r = b1.reshape(1, _D)
    b2r = b2.reshape(1, _D)
    b3r = b3.reshape(1, _D)
    bfcr = bfc.reshape(1, _DO)

    t1 = _mm(x, W1)
    s1 = _segsum(t1, src2, dst2)
    t2 = _fused(t1, s1[:_N], s1[_NP:_NP + _N], b1r, W2)
    s2 = _segsum(t2, src2, dst2)
    t3 = _fused(t2, s2[:_N], s2[_NP:_NP + _N], b2r, W3)
    s3 = _segsum(t3, src2, dst2)
    return _pool(t3, s3[:_N], s3[_NP:_NP + _N], b3r, batch2, Wfc, bfcr)


# speedup vs baseline: 1.2674x; 1.0218x over previous
import jax
import jax.numpy as jnp
from jax import lax
from jax.experimental import pallas as pl
from jax.experimental.pallas import tpu as pltpu
from jax.experimental.pallas import tpu_sc as plsc

_N = 10000
_E = 320000
_D = 128
_G = 64
_DO = 16

_NC = 2
_NS = 16
_NW = _NC * _NS
_K = 80
_CH = 125
_EPW = _CH * _K
_NP = 10240
_RPT = _NP // _NS


def _segsum_body(t_hbm, s_hbm, d_hbm, out_hbm, didx, sidx, rows0, rows1,
                 acc, dsem, g0sem, g1sem):
    c = lax.axis_index("c")
    s = lax.axis_index("s")
    wid = c * _NS + s

    cp_d = pltpu.async_copy(d_hbm.at[wid], didx, dsem)
    cp_s = pltpu.async_copy(s_hbm.at[wid], sidx, dsem)

    def zrow(r, carry):
        for j in range(_D // 16):
            rows0[r, pl.ds(j * 16, 16)] = jnp.zeros((16,), jnp.float32)
        return carry
    lax.fori_loop(0, _K, zrow, 0)
    for j in range(_RPT // _K):
        pltpu.sync_copy(rows0, acc.at[pl.ds(s * _RPT + j * _K, _K)])
    cp_d.wait()
    cp_s.wait()
    plsc.subcore_barrier()

    pltpu.async_copy(t_hbm.at[sidx.at[pl.ds(0, _K)]], rows0, g0sem)

    def body(j, carry):
        i0 = 2 * j
        i1 = 2 * j + 1

        pltpu.async_copy(t_hbm.at[sidx.at[pl.ds(i1 * _K, _K)]], rows1, g1sem)
        pltpu.make_async_copy(t_hbm.at[sidx.at[pl.ds(i0 * _K, _K)]], rows0,
                              g0sem).wait()
        pltpu.sync_copy(rows0, acc.at[didx.at[i0]], add=True)

        @pl.when(i0 + 2 < _CH)
        def _g0():
            pltpu.async_copy(t_hbm.at[sidx.at[pl.ds((i0 + 2) * _K, _K)]],
                             rows0, g0sem)

        pltpu.make_async_copy(t_hbm.at[sidx.at[pl.ds(i1 * _K, _K)]], rows1,
                              g1sem).wait()
        pltpu.sync_copy(rows1, acc.at[didx.at[i1]], add=True)

        return carry
    lax.fori_loop(0, _CH // 2, body, 0)

    last = _CH - 1
    pltpu.make_async_copy(t_hbm.at[sidx.at[pl.ds(last * _K, _K)]], rows0,
                          g0sem).wait()
    pltpu.sync_copy(rows0, acc.at[didx.at[last]], add=True)

    plsc.subcore_barrier()
    pltpu.sync_copy(acc.at[pl.ds(s * _RPT, _RPT)],
                    out_hbm.at[pl.ds(c * _NP + s * _RPT, _RPT)])


_segsum = pl.kernel(
    _segsum_body,
    out_type=jax.ShapeDtypeStruct((_NC * _NP, _D), jnp.float32),
    mesh=plsc.VectorSubcoreMesh(core_axis_name="c", subcore_axis_name="s"),
    scratch_types=[
        pltpu.VMEM((_CH, _K), jnp.int32),
        pltpu.VMEM((_EPW,), jnp.int32),
        pltpu.VMEM((_K, _D), jnp.float32),
        pltpu.VMEM((_K, _D), jnp.float32),
        pltpu.VMEM_SHARED((_NP, _D), jnp.float32),
        pltpu.SemaphoreType.DMA,
        pltpu.SemaphoreType.DMA,
        pltpu.SemaphoreType.DMA,
    ],
)

_BLK = 1000


def _combine_body(t_ref, p0_ref, p1_ref, w_ref, b_ref, o_ref):
    z = t_ref[...] + p0_ref[...] + p1_ref[...]
    o_ref[...] = jnp.maximum(
        jnp.dot(z, w_ref[...], preferred_element_type=jnp.float32)
        + b_ref[...], 0.0)


def _combine(t, p0, p1, w, b):
    return pl.pallas_call(
        _combine_body,
        grid=(_N // _BLK,),
        in_specs=[pl.BlockSpec((_BLK, _D), lambda i: (i, 0)),
                  pl.BlockSpec((_BLK, _D), lambda i: (i, 0)),
                  pl.BlockSpec((_BLK, _D), lambda i: (i, 0)),
                  pl.BlockSpec((_D, _D), lambda i: (0, 0)),
                  pl.BlockSpec((1, _D), lambda i: (0, 0))],
        out_specs=pl.BlockSpec((_BLK, _D), lambda i: (i, 0)),
        out_shape=jax.ShapeDtypeStruct((_N, _D), jnp.float32),
    )(t, p0, p1, w, b)


def _pool_body(t_ref, p0_ref, p1_ref, w_ref, b_ref, batch_ref, wfc_ref,
               bfc_ref, o_ref, sums, cnt):
    i = pl.program_id(0)

    @pl.when(i == 0)
    def _init():
        sums[...] = jnp.zeros_like(sums)
        cnt[...] = jnp.zeros_like(cnt)

    z = t_ref[...] + p0_ref[...] + p1_ref[...]
    h = jnp.maximum(
        jnp.dot(z, w_ref[...], preferred_element_type=jnp.float32)
        + b_ref[...], 0.0)
    gids = lax.broadcasted_iota(jnp.int32, (_BLK, _G), 1)
    oh = (batch_ref[...] == gids).astype(jnp.float32)
    sums[...] += lax.dot_general(oh, h, (((0,), (0,)), ((), ())),
                                 preferred_element_type=jnp.float32)
    cnt[...] += lax.dot_general(oh, jnp.ones((_BLK, _D), jnp.float32),
                                (((0,), (0,)), ((), ())),
                                preferred_element_type=jnp.float32)

    @pl.when(i == _N // _BLK - 1)
    def _fin():
        pooled = sums[...] / jnp.maximum(cnt[...], 1.0)
        o_ref[...] = jnp.dot(pooled, wfc_ref[...],
                             preferred_element_type=jnp.float32) + bfc_ref[...]


def _pool(t, p0, p1, w, b, batch2, wfc, bfc):
    return pl.pallas_call(
        _pool_body,
        grid=(_N // _BLK,),
        in_specs=[pl.BlockSpec((_BLK, _D), lambda i: (i, 0)),
                  pl.BlockSpec((_BLK, _D), lambda i: (i, 0)),
                  pl.BlockSpec((_BLK, _D), lambda i: (i, 0)),
                  pl.BlockSpec((_D, _D), lambda i: (0, 0)),
                  pl.BlockSpec((1, _D), lambda i: (0, 0)),
                  pl.BlockSpec((_BLK, 1), lambda i: (i, 0)),
                  pl.BlockSpec((_D, _DO), lambda i: (0, 0)),
                  pl.BlockSpec((1, _DO), lambda i: (0, 0))],
        out_specs=pl.BlockSpec((_G, _DO), lambda i: (0, 0)),
        out_shape=jax.ShapeDtypeStruct((_G, _DO), jnp.float32),
        scratch_shapes=[pltpu.VMEM((_G, _D), jnp.float32),
                        pltpu.VMEM((_G, _D), jnp.float32)],
    )(t, p0, p1, w, b, batch2, wfc, bfc)


def kernel(x, edge_index, batch, W1, b1, W2, b2, W3, b3, Wfc, bfc):
    src2 = edge_index[0].reshape(_NW, _EPW)
    dst2 = edge_index[1].reshape(_NW, _CH, _K)
    batch2 = batch.reshape(_N, 1)
    b1r = b1.reshape(1, _D)
    b2r = b2.reshape(1, _D)
    b3r = b3.reshape(1, _D)
    bfcr = bfc.reshape(1, _DO)

    s1 = _segsum(x, src2, dst2)
    h1 = _combine(x, s1[:_N], s1[_NP:_NP + _N], W1, b1r)
    s2 = _segsum(h1, src2, dst2)
    h2 = _combine(h1, s2[:_N], s2[_NP:_NP + _N], W2, b2r)
    s3 = _segsum(h2, src2, dst2)
    return _pool(h2, s3[:_N], s3[_NP:_NP + _N], W3, b3r, batch2, Wfc, bfcr)
